# baseline (speedup 1.0000x reference)
import functools

import jax
import jax.numpy as jnp
from jax import lax
from jax.experimental import pallas as pl
from jax.experimental.pallas import tpu as pltpu
from jax.experimental.pallas import tpu_sc as plsc

_NC = 2
_NS = 16
_IT = 2048


@functools.cache
def _make_emajor(V, D, S, N):
    e_per_c = D // _NC
    hr_tot = S * (N // _IT)
    hr_per_w = hr_tot // _NS
    tok_per_w = hr_per_w * _IT
    n_pairs = (hr_per_w - 1) // 2
    seg = ((V // _NS) + 31) // 32 * 32
    seg_tail = V - (_NS - 1) * seg
    assert seg % 8 == 0 and seg_tail % 8 == 0 and seg_tail > 0
    mesh = plsc.VectorSubcoreMesh(core_axis_name="c", subcore_axis_name="s")

    @functools.partial(
        pl.kernel,
        out_type=jax.ShapeDtypeStruct((S, D, N), jnp.float32),
        mesh=mesh,
        scratch_types=[
            pltpu.VMEM_SHARED((V,), jnp.float32),
            pltpu.VMEM((tok_per_w,), jnp.int32),
            pltpu.VMEM((_IT,), jnp.float32),
            pltpu.VMEM((_IT,), jnp.float32),
            pltpu.SemaphoreType.DMA,
            pltpu.SemaphoreType.DMA((2,)),
            pltpu.SemaphoreType.DMA((2,)),
        ],
        compiler_params=pltpu.CompilerParams(use_tc_tiling_on_sc=False),
    )
    def emajor_kernel(wt_hbm, tok_hbm, out_hbm, row_v, tok_v, val0, val1,
                      ssem, gsem, wsem):
        sid = lax.axis_index("s")
        cid = lax.axis_index("c")
        vals = (val0, val1)

        t0 = pl.multiple_of(sid * tok_per_w, 8)
        pltpu.sync_copy(tok_hbm.at[pl.ds(t0, tok_per_w)], tok_v)

        def idx_ref(i):
            return tok_v.at[pl.ds(i * _IT, _IT)]

        def fire_gather(i, b):
            pltpu.async_copy(row_v.at[idx_ref(i)], vals[b], gsem.at[b])

        def wait_gather(i, b):
            pltpu.make_async_copy(row_v.at[idx_ref(i)], vals[b],
                                  gsem.at[b]).wait()

        def out_slice(i, e):
            hr = sid * hr_per_w + i
            return out_hbm.at[hr // 2, e, pl.ds((hr % 2) * _IT, _IT)]

        def fire_write(i, b, e):
            pltpu.async_copy(vals[b], out_slice(i, e), wsem.at[b])

        def drain_write(i, b, e):
            pltpu.make_async_copy(vals[b], out_slice(i, e), wsem.at[b]).wait()

        def e_body(el, carry):
            e = cid * e_per_c + el
            plsc.subcore_barrier()
            soff = pl.multiple_of(sid * seg, 8)

            @pl.when(sid < _NS - 1)
            def _():
                pltpu.async_copy(wt_hbm.at[pl.ds(e * V + soff, seg)],
                                 row_v.at[pl.ds(soff, seg)], ssem).wait()

            @pl.when(sid == _NS - 1)
            def _():
                pltpu.async_copy(wt_hbm.at[pl.ds(e * V + soff, seg_tail)],
                                 row_v.at[pl.ds(soff, seg_tail)], ssem).wait()
            plsc.subcore_barrier()

            @pl.when(el > 0)
            def _():
                drain_write(0, 0, e)
            fire_gather(0, 0)

            def pair(j, c2):
                i0 = 2 * j
                wait_gather(i0, 0)

                @pl.when((el > 0) | (j > 0))
                def _():
                    drain_write(0, 1, e)
                fire_gather(i0 + 1, 1)
                fire_write(i0, 0, e)
                wait_gather(i0 + 1, 1)
                drain_write(0, 0, e)
                fire_gather(i0 + 2, 0)
                fire_write(i0 + 1, 1, e)
                return c2

            lax.fori_loop(0, n_pairs, pair, 0)
            wait_gather(hr_per_w - 1, 0)
            fire_write(hr_per_w - 1, 0, e)
            return carry

        lax.fori_loop(0, e_per_c, e_body, 0)
        drain_write(0, 0, 0)
        drain_write(0, 1, 0)

    return emajor_kernel


def kernel(token_ids, weight):
    bsz, seq = token_ids.shape
    v, d = weight.shape
    wt_lin = weight.T.reshape(-1)
    toks = token_ids.T.reshape(-1)
    out_phys = _make_emajor(v, d, seq, bsz)(wt_lin, toks)
    return jnp.transpose(out_phys, (2, 0, 1))


# speedup vs baseline: 4.8044x; 4.8044x over previous
import functools

import jax
import jax.numpy as jnp
from jax import lax
from jax.experimental import pallas as pl
from jax.experimental.pallas import tpu as pltpu
from jax.experimental.pallas import tpu_sc as plsc

_NUM_CORES = 2
_NUM_SUBCORES = 16
_CHUNK = 512


@functools.cache
def _make_gather(B, D):
    nw = _NUM_CORES * _NUM_SUBCORES
    assert B % (8 * nw) == 0
    b_per_w = B // nw
    assert b_per_w % (2 * _CHUNK) == 0
    n_loops = b_per_w // (2 * _CHUNK)
    mesh = plsc.VectorSubcoreMesh(core_axis_name="c", subcore_axis_name="s")

    @functools.partial(
        pl.kernel,
        out_type=jax.ShapeDtypeStruct((B, D), jnp.float32),
        mesh=mesh,
        scratch_types=[
            pltpu.VMEM((b_per_w,), jnp.int32),
            pltpu.VMEM((2, _CHUNK, D), jnp.float32),
            pltpu.SemaphoreType.DMA((2,)),
        ],
        compiler_params=pltpu.CompilerParams(use_tc_tiling_on_sc=False),
    )
    def gather_kernel(idx_hbm, table_hbm, out_hbm, idx_v, rows_v, gsem):
        wid = lax.axis_index("s") * _NUM_CORES + lax.axis_index("c")
        base = pl.multiple_of(wid * b_per_w, 8)

        pltpu.sync_copy(idx_hbm.at[pl.ds(base, b_per_w)], idx_v)

        def start_gather(c, b):
            loc = pl.multiple_of(c * _CHUNK, 8)
            pltpu.async_copy(
                table_hbm.at[idx_v.at[pl.ds(loc, _CHUNK)]], rows_v.at[b],
                gsem.at[b])

        def wait_gather(c, b):
            loc = pl.multiple_of(c * _CHUNK, 8)
            pltpu.make_async_copy(
                table_hbm.at[idx_v.at[pl.ds(loc, _CHUNK)]], rows_v.at[b],
                gsem.at[b]).wait()

        start_gather(0, 0)

        def body(j, carry):
            for b in (0, 1):
                c = 2 * j + b
                wait_gather(c, b)
                if b == 0:
                    start_gather(c + 1, 1)
                else:

                    @pl.when(j < n_loops - 1)
                    def _():
                        start_gather(c + 1, 0)

                glob = pl.multiple_of(base + c * _CHUNK, 8)
                pltpu.sync_copy(rows_v.at[b], out_hbm.at[pl.ds(glob, _CHUNK)])
            return carry

        lax.fori_loop(0, n_loops, body, 0)

    return gather_kernel


def kernel(token_ids, weight):
    bsz, seq = token_ids.shape
    _, d = weight.shape
    flat = token_ids.reshape(bsz * seq).astype(jnp.int32)
    out = _make_gather(bsz * seq, d)(flat, weight)
    return out.reshape(bsz, seq, d)
